# baseline (speedup 1.0000x reference)
import functools

import jax
import jax.numpy as jnp
import numpy as np
from jax.experimental import pallas as pl
from jax.experimental.pallas import tpu as pltpu

BASE_AMP = 0.08


def _anchor_starts_np(signal_len, num_segments, seg_length):
    max_start = max(signal_len - seg_length, 0)
    head = 0.1 * signal_len
    tail = max(0.0, 0.78 * signal_len)
    anchors = np.linspace(head, tail, num_segments)
    return np.clip(np.round(anchors), 0, max_start).astype(np.int64)


def _fused_kernel(x_ref, pi_ref, pq_ref, scale_ref, out_ref, row_ref, *,
                  starts, seg_length, signal_len, inv_n):
    @pl.when(pl.program_id(0) == 0)
    def _build_row():
        pi = jnp.tanh(pi_ref[0, :])
        pq = jnp.tanh(pq_ref[0, :])
        row_ref[...] = jnp.zeros((1, 2, signal_len), dtype=jnp.float32)
        for k, s in enumerate(starts):
            sc = jax.nn.relu(scale_ref[k])
            row_ref[0, 0, pl.ds(s, seg_length)] = sc * pi
            row_ref[0, 1, pl.ds(s, seg_length)] = sc * pq

    xb = x_ref[...]
    ss = jnp.sum(xb * xb, axis=(1, 2), keepdims=True)
    amp = BASE_AMP * jnp.sqrt(ss * inv_n + 1e-12)
    out_ref[...] = xb + amp * row_ref[...]


def kernel(x, pattern_i, pattern_q, segment_scale):
    batch, ch, signal_len = x.shape
    seg_length = pattern_i.shape[0]
    num_segments = segment_scale.shape[0]
    starts = [int(s) for s in
              _anchor_starts_np(signal_len, num_segments, seg_length)]

    bb = 64
    grid = (batch // bb,)

    body = functools.partial(
        _fused_kernel, starts=starts, seg_length=seg_length,
        signal_len=signal_len, inv_n=1.0 / (ch * signal_len))

    return pl.pallas_call(
        body,
        grid=grid,
        in_specs=[
            pl.BlockSpec((bb, ch, signal_len), lambda i: (i, 0, 0)),
            pl.BlockSpec((1, seg_length), lambda i: (0, 0)),
            pl.BlockSpec((1, seg_length), lambda i: (0, 0)),
            pl.BlockSpec(memory_space=pltpu.SMEM),
        ],
        out_specs=pl.BlockSpec((bb, ch, signal_len), lambda i: (i, 0, 0)),
        out_shape=jax.ShapeDtypeStruct((batch, ch, signal_len), jnp.float32),
        scratch_shapes=[pltpu.VMEM((1, ch, signal_len), jnp.float32)],
    )(x, pattern_i[None, :], pattern_q[None, :], segment_scale)


# speedup vs baseline: 1.2306x; 1.2306x over previous
import functools

import jax
import jax.numpy as jnp
import numpy as np
from jax.experimental import pallas as pl
from jax.experimental.pallas import tpu as pltpu

BASE_AMP = 0.08


def _anchor_starts_np(signal_len, num_segments, seg_length):
    max_start = max(signal_len - seg_length, 0)
    head = 0.1 * signal_len
    tail = max(0.0, 0.78 * signal_len)
    anchors = np.linspace(head, tail, num_segments)
    return np.clip(np.round(anchors), 0, max_start).astype(np.int64)


def _fused_kernel(x_ref, pi_ref, pq_ref, scale_ref, out_ref, row_ref, *,
                  starts, seg_length, signal_len, inv_n):
    @pl.when(pl.program_id(0) == 0)
    def _build_row():
        pi = jnp.tanh(pi_ref[0, :])
        pq = jnp.tanh(pq_ref[0, :])
        row_ref[...] = jnp.zeros((2, signal_len), dtype=jnp.float32)
        for k, s in enumerate(starts):
            sc = jax.nn.relu(scale_ref[k])
            row_ref[0, pl.ds(s, seg_length)] = sc * pi
            row_ref[1, pl.ds(s, seg_length)] = sc * pq

    x0 = x_ref[:, 0, :]
    x1 = x_ref[:, 1, :]
    a = x0 * x0 + x1 * x1
    ss = jnp.sum(a, axis=1, keepdims=True)
    amp = BASE_AMP * jnp.sqrt(ss * inv_n + 1e-12)
    out_ref[...] = x_ref[...] + amp[:, :, None] * row_ref[...][None, :, :]


def kernel(x, pattern_i, pattern_q, segment_scale):
    batch, ch, signal_len = x.shape
    seg_length = pattern_i.shape[0]
    num_segments = segment_scale.shape[0]
    starts = [int(s) for s in
              _anchor_starts_np(signal_len, num_segments, seg_length)]

    bb = 128
    grid = (batch // bb,)

    body = functools.partial(
        _fused_kernel, starts=starts, seg_length=seg_length,
        signal_len=signal_len, inv_n=1.0 / (ch * signal_len))

    return pl.pallas_call(
        body,
        grid=grid,
        in_specs=[
            pl.BlockSpec((bb, ch, signal_len), lambda i: (i, 0, 0)),
            pl.BlockSpec((1, seg_length), lambda i: (0, 0)),
            pl.BlockSpec((1, seg_length), lambda i: (0, 0)),
            pl.BlockSpec(memory_space=pltpu.SMEM),
        ],
        out_specs=pl.BlockSpec((bb, ch, signal_len), lambda i: (i, 0, 0)),
        out_shape=jax.ShapeDtypeStruct((batch, ch, signal_len), jnp.float32),
        scratch_shapes=[pltpu.VMEM((ch, signal_len), jnp.float32)],
    )(x, pattern_i[None, :], pattern_q[None, :], segment_scale)
